# baseline (speedup 1.0000x reference)
import jax
import jax.numpy as jnp
from jax import lax
from jax.experimental import pallas as pl
from jax.experimental.pallas import tpu as pltpu
from jax.experimental.pallas import tpu_sc as plsc

B = 16384
D = 64
NC = 2
NS = 16
NW = NC * NS
ROWS_PER_W = B // NW
CHUNK = 128
NCHUNK = ROWS_PER_W // CHUNK
NLANE = 16
DCH = D // NLANE


def _rsqrt_newton(x):
    i = lax.bitcast_convert_type(x, jnp.int32)
    i = jnp.int32(0x5F3759DF) - lax.shift_right_logical(i, 1)
    y = lax.bitcast_convert_type(i, jnp.float32)
    for _ in range(3):
        y = y * (1.5 - 0.5 * x * y * y)
    return y


def _lane_sum(v):
    lanes = lax.iota(jnp.int32, NLANE)
    for s in (1, 2, 4, 8):
        v = v + v.at[lanes ^ s].get(mode="promise_in_bounds",
                                    unique_indices=True)
    return v


def _sc_body(emb_hbm, transl_hbm, rtr_hbm, etr_hbm, opidx_hbm, entidx_hbm,
             out_hbm, idx_e, idx_o, emb_v, ent_v, rtr_v, rem_v, out_v,
             sem0, sem1, sem2, sem3):
    wid = lax.axis_index("s") * NC + lax.axis_index("c")
    wbase = wid * ROWS_PER_W

    for c in range(NCHUNK):
        base = wbase + c * CHUNK
        pltpu.sync_copy(entidx_hbm.at[pl.ds(base, CHUNK)], idx_e)
        pltpu.sync_copy(opidx_hbm.at[pl.ds(base, CHUNK)], idx_o)
        cp0 = pltpu.async_copy(etr_hbm.at[idx_e], ent_v, sem0)
        cp1 = pltpu.async_copy(transl_hbm.at[idx_o], rem_v, sem1)
        cp2 = pltpu.async_copy(rtr_hbm.at[idx_o], rtr_v, sem2)
        cp3 = pltpu.async_copy(emb_hbm.at[pl.ds(base, CHUNK)], emb_v, sem3)
        cp0.wait()
        cp1.wait()
        cp2.wait()
        cp3.wait()

        def row(r, _):
            e = [emb_v[r, pl.ds(j * NLANE, NLANE)] for j in range(DCH)]
            t = [ent_v[r, pl.ds(j * NLANE, NLANE)] for j in range(DCH)]
            acc = e[0] * t[0]
            for j in range(1, DCH):
                acc = acc + e[j] * t[j]
            dot = _lane_sum(acc)
            x = []
            ss = None
            for j in range(DCH):
                xj = (e[j] + dot * rtr_v[r, pl.ds(j * NLANE, NLANE)]
                      + rem_v[r, pl.ds(j * NLANE, NLANE)])
                x.append(xj)
                ss = xj * xj if ss is None else ss + xj * xj
            rinv = _rsqrt_newton(_lane_sum(ss))
            for j in range(DCH):
                out_v[r, pl.ds(j * NLANE, NLANE)] = x[j] * rinv
            return 0

        lax.fori_loop(0, CHUNK, row, 0)
        pltpu.sync_copy(out_v, out_hbm.at[pl.ds(base, CHUNK)])


def kernel(embeddings, translations, relation_transfer, entity_transfer,
           operator_idxs, entity_list, relation_dim, entity_dim, flag, rel_id):
    del relation_dim, entity_dim, flag, rel_id
    mesh = plsc.VectorSubcoreMesh(core_axis_name="c", subcore_axis_name="s")
    f = pl.kernel(
        _sc_body,
        out_type=jax.ShapeDtypeStruct((B, D), jnp.float32),
        mesh=mesh,
        compiler_params=pltpu.CompilerParams(use_tc_tiling_on_sc=False),
        scratch_types=[
            pltpu.VMEM((CHUNK,), jnp.int32),
            pltpu.VMEM((CHUNK,), jnp.int32),
            pltpu.VMEM((CHUNK, D), jnp.float32),
            pltpu.VMEM((CHUNK, D), jnp.float32),
            pltpu.VMEM((CHUNK, D), jnp.float32),
            pltpu.VMEM((CHUNK, D), jnp.float32),
            pltpu.VMEM((CHUNK, D), jnp.float32),
            pltpu.SemaphoreType.DMA,
            pltpu.SemaphoreType.DMA,
            pltpu.SemaphoreType.DMA,
            pltpu.SemaphoreType.DMA,
        ],
    )
    return f(embeddings, translations, relation_transfer, entity_transfer,
             operator_idxs, entity_list)


# speedup vs baseline: 1.6467x; 1.6467x over previous
import jax
import jax.numpy as jnp
from jax import lax
from jax.experimental import pallas as pl
from jax.experimental.pallas import tpu as pltpu
from jax.experimental.pallas import tpu_sc as plsc

B = 16384
D = 64
NOPS = 1000
NC = 2
NS = 16
NW = NC * NS
ROWS_PER_W = B // NW
CHUNK = 128
NCHUNK = ROWS_PER_W // CHUNK
NLANE = 16
DCH = D // NLANE


def _rsqrt_newton(x):
    i = lax.bitcast_convert_type(x, jnp.int32)
    i = jnp.int32(0x5F3759DF) - lax.shift_right_logical(i, 1)
    y = lax.bitcast_convert_type(i, jnp.float32)
    for _ in range(3):
        y = y * (1.5 - 0.5 * x * y * y)
    return y


def _lane_sum(v):
    lanes = lax.iota(jnp.int32, NLANE)
    for s in (1, 2, 4, 8):
        v = v + v.at[lanes ^ s].get(mode="promise_in_bounds",
                                    unique_indices=True)
    return v


def _sc_body(emb_hbm, transl_hbm, rtr_hbm, etr_hbm, opidx_hbm, entidx_hbm,
             out_hbm, transl_s, rtr_s, idx_e, idx_o, emb_v, ent_v,
             rtr_v, rem_v, out_v, sem_e, sem_g, sem_s):
    cid = lax.axis_index("c")
    sid = lax.axis_index("s")
    wid = sid * NC + cid
    wbase = wid * ROWS_PER_W

    @pl.when(sid == 0)
    def _():
        pltpu.sync_copy(transl_hbm, transl_s)
        pltpu.sync_copy(rtr_hbm, rtr_s)

    plsc.subcore_barrier()

    def chunk(c, _):
        base = wbase + c * CHUNK
        pltpu.sync_copy(entidx_hbm.at[pl.ds(base, CHUNK)], idx_e)
        pltpu.sync_copy(opidx_hbm.at[pl.ds(base, CHUNK)], idx_o)
        cps = []
        for g in range(CHUNK // NLANE):
            idx16 = idx_e[pl.ds(g * NLANE, NLANE)]
            for k in range(NLANE):
                r = g * NLANE + k
                cps.append(pltpu.async_copy(
                    etr_hbm.at[pl.ds(idx16[k], 1)],
                    ent_v.at[pl.ds(r, 1)], sem_e))
        cp1 = pltpu.async_copy(transl_s.at[idx_o], rem_v, sem_g)
        cp2 = pltpu.async_copy(rtr_s.at[idx_o], rtr_v, sem_g)
        cp3 = pltpu.async_copy(emb_hbm.at[pl.ds(base, CHUNK)], emb_v, sem_s)
        for cp in cps:
            cp.wait()
        cp1.wait()
        cp2.wait()
        cp3.wait()

        def grp(g, _):
            for k in range(NLANE):
                r = g * NLANE + k
                e = [emb_v[r, pl.ds(j * NLANE, NLANE)] for j in range(DCH)]
                t = [ent_v[r, pl.ds(j * NLANE, NLANE)] for j in range(DCH)]
                acc = e[0] * t[0]
                for j in range(1, DCH):
                    acc = acc + e[j] * t[j]
                dot = _lane_sum(acc)
                x = []
                ss = None
                for j in range(DCH):
                    xj = (e[j] + dot * rtr_v[r, pl.ds(j * NLANE, NLANE)]
                          + rem_v[r, pl.ds(j * NLANE, NLANE)])
                    x.append(xj)
                    ss = xj * xj if ss is None else ss + xj * xj
                rinv = _rsqrt_newton(_lane_sum(ss))
                for j in range(DCH):
                    out_v[r, pl.ds(j * NLANE, NLANE)] = x[j] * rinv
            return 0

        lax.fori_loop(0, CHUNK // NLANE, grp, 0)
        pltpu.sync_copy(out_v, out_hbm.at[pl.ds(base, CHUNK)])
        return 0

    lax.fori_loop(0, NCHUNK, chunk, 0)


def kernel(embeddings, translations, relation_transfer, entity_transfer,
           operator_idxs, entity_list, relation_dim, entity_dim, flag, rel_id):
    del relation_dim, entity_dim, flag, rel_id
    mesh = plsc.VectorSubcoreMesh(core_axis_name="c", subcore_axis_name="s")
    f = pl.kernel(
        _sc_body,
        out_type=jax.ShapeDtypeStruct((B, D), jnp.float32),
        mesh=mesh,
        scratch_types=[
            pltpu.VMEM_SHARED((NOPS, D), jnp.float32),
            pltpu.VMEM_SHARED((NOPS, D), jnp.float32),
            pltpu.VMEM((CHUNK,), jnp.int32),
            pltpu.VMEM((CHUNK,), jnp.int32),
            pltpu.VMEM((CHUNK, D), jnp.float32),
            pltpu.VMEM((CHUNK, D), jnp.float32),
            pltpu.VMEM((CHUNK, D), jnp.float32),
            pltpu.VMEM((CHUNK, D), jnp.float32),
            pltpu.VMEM((CHUNK, D), jnp.float32),
            pltpu.SemaphoreType.DMA,
            pltpu.SemaphoreType.DMA,
            pltpu.SemaphoreType.DMA,
        ],
    )
    return f(embeddings, translations, relation_transfer, entity_transfer,
             operator_idxs, entity_list)
